# baseline (speedup 1.0000x reference)
import functools

import jax
import jax.numpy as jnp
from jax import lax
from jax.experimental import pallas as pl
from jax.experimental.pallas import tpu as pltpu
from jax.experimental.pallas import tpu_sc as plsc

NN = 10000
NP = 10240
PN = NN
E = 320000
NCLS = 40
H = 16
CPAD = 48
NC = 2
NS = 16
NT = NC * NS
CHUNK = 128
GPT = 79
EP = NT * GPT * CHUNK
DEGR = 80
NGRAPH = 250
NSLOT = 256
SLOT_SH = 18


def _mesh():
    return plsc.VectorSubcoreMesh(core_axis_name="c", subcore_axis_name="s")


def _make_edge_acc(width):
    rpt = NP // NS

    @functools.partial(
        pl.kernel,
        out_type=jax.ShapeDtypeStruct((NC, NP, width), jnp.float32),
        mesh=_mesh(),
        scratch_types=[
            pltpu.VMEM((GPT, CHUNK), jnp.int32),
            pltpu.VMEM((GPT, CHUNK), jnp.int32),
            pltpu.VMEM((CHUNK, width), jnp.float32),
            pltpu.VMEM((CHUNK, width), jnp.float32),
            pltpu.VMEM_SHARED((NP, width), jnp.float32),
            pltpu.SemaphoreType.DMA,
            pltpu.SemaphoreType.DMA,
        ],
        compiler_params=pltpu.CompilerParams(use_tc_tiling_on_sc=False),
    )
    def k(src_hbm, dst_hbm, feat_hbm, zrows_hbm, out_hbm,
          src_v, dst_v, rows_a, rows_b, acc_sh, sem_a, sem_b):
        cid = lax.axis_index("c")
        sid = lax.axis_index("s")
        wid = sid * NC + cid
        pltpu.sync_copy(zrows_hbm, acc_sh.at[pl.ds(sid * rpt, rpt)])
        pltpu.sync_copy(src_hbm.at[wid], src_v)
        pltpu.sync_copy(dst_hbm.at[wid], dst_v)
        plsc.subcore_barrier()
        pltpu.async_copy(feat_hbm.at[src_v.at[0]], rows_a, sem_a)

        def body(i, carry):
            ga = 2 * i
            pltpu.async_copy(feat_hbm.at[src_v.at[ga + 1]], rows_b, sem_b)
            pltpu.make_async_copy(feat_hbm.at[src_v.at[ga]], rows_a, sem_a).wait()
            pltpu.sync_copy(rows_a, acc_sh.at[dst_v.at[ga]], add=True)
            pltpu.async_copy(feat_hbm.at[src_v.at[ga + 2]], rows_a, sem_a)
            pltpu.make_async_copy(feat_hbm.at[src_v.at[ga + 1]], rows_b, sem_b).wait()
            pltpu.sync_copy(rows_b, acc_sh.at[dst_v.at[ga + 1]], add=True)
            return carry

        lax.fori_loop(0, (GPT - 1) // 2, body, 0)
        pltpu.make_async_copy(feat_hbm.at[src_v.at[GPT - 1]], rows_a, sem_a).wait()
        pltpu.sync_copy(rows_a, acc_sh.at[dst_v.at[GPT - 1]], add=True)
        plsc.subcore_barrier()
        pltpu.sync_copy(acc_sh.at[pl.ds(sid * rpt, rpt)],
                        out_hbm.at[cid].at[pl.ds(sid * rpt, rpt)])

    return k


def _make_center_acc():
    @functools.partial(
        pl.kernel,
        out_type=jax.ShapeDtypeStruct((NC, NS, NSLOT, CPAD), jnp.float32),
        mesh=_mesh(),
        scratch_types=[
            pltpu.VMEM((16,), jnp.int32),
            pltpu.VMEM((CHUNK,), jnp.int32),
            pltpu.VMEM((CHUNK,), jnp.int32),
            pltpu.VMEM((CHUNK,), jnp.int32),
            pltpu.VMEM((CHUNK, CPAD), jnp.float32),
            pltpu.VMEM_SHARED((NS, NSLOT, CPAD), jnp.float32),
            pltpu.SemaphoreType.DMA,
        ],
        compiler_params=pltpu.CompilerParams(
            use_tc_tiling_on_sc=False, needs_layout_passes=False),
    )
    def k(pk_hbm, cnt_hbm, feat_hbm, zrows_hbm, out_hbm,
          cnt_v, pkg_v, gidx_v, sidx_v, rows_v, acc_sh, sem):
        cid = lax.axis_index("c")
        sid = lax.axis_index("s")
        wid = sid * NC + cid
        pltpu.sync_copy(zrows_hbm.at[pl.ds(0, NSLOT)], acc_sh.at[sid])
        pltpu.sync_copy(cnt_hbm.at[wid], cnt_v)
        cnt = cnt_v[pl.ds(0, 16)][0]
        ngroups = lax.div(cnt + CHUNK - 1, CHUNK)

        def group(g, carry):
            pltpu.sync_copy(pk_hbm.at[wid].at[g], pkg_v)
            for j in range(CHUNK // 16):
                p = pkg_v[pl.ds(j * 16, 16)]
                gidx_v[pl.ds(j * 16, 16)] = jnp.bitwise_and(p, (1 << SLOT_SH) - 1)
                sidx_v[pl.ds(j * 16, 16)] = lax.shift_right_logical(p, SLOT_SH)
            pltpu.async_copy(feat_hbm.at[gidx_v], rows_v, sem).wait()
            pltpu.sync_copy(rows_v, acc_sh.at[sid].at[sidx_v], add=True)
            return carry

        lax.fori_loop(0, ngroups, group, 0)
        pltpu.sync_copy(acc_sh.at[sid], out_hbm.at[cid].at[sid])

    return k


def _make_deg_compact():
    rpt = NP // NS
    EPW = GPT * CHUNK
    PADW = PN | (NGRAPH << SLOT_SH)

    @functools.partial(
        pl.kernel,
        out_type=(jax.ShapeDtypeStruct((NC, NP), jnp.float32),
                  jax.ShapeDtypeStruct((NT, GPT + 1, CHUNK), jnp.int32),
                  jax.ShapeDtypeStruct((NT, 16), jnp.int32)),
        mesh=_mesh(),
        scratch_types=[
            pltpu.VMEM((EPW,), jnp.int32),
            pltpu.VMEM((EPW,), jnp.int32),
            pltpu.VMEM((NP,), jnp.int32),
            pltpu.VMEM((EPW + CHUNK,), jnp.int32),
            pltpu.VMEM((16,), jnp.int32),
            pltpu.VMEM((NP,), jnp.float32),
            pltpu.VMEM((rpt,), jnp.float32),
            pltpu.VMEM((rpt,), jnp.float32),
            pltpu.VMEM_SHARED((NS, NP), jnp.float32),
        ],
        compiler_params=pltpu.CompilerParams(
            use_tc_tiling_on_sc=False, needs_layout_passes=False),
    )
    def k(src_hbm, dst_hbm, slotmap_hbm, zdeg_hbm,
          deg_hbm, pk_hbm, cnt_hbm,
          src_v, dst_v, slot_v, pk_v, cnt_v, deg_l, acc_v, tmp_v, stage_sh):
        cid = lax.axis_index("c")
        sid = lax.axis_index("s")
        wid = sid * NC + cid
        pltpu.sync_copy(zdeg_hbm, deg_l)
        pltpu.sync_copy(src_hbm.at[wid], src_v)
        pltpu.sync_copy(dst_hbm.at[wid], dst_v)
        pltpu.sync_copy(slotmap_hbm, slot_v)
        ones = jnp.full((16,), 1.0, jnp.float32)

        def body(i, cnt):
            off = pl.multiple_of(i * 16, 16)
            dv = dst_v[pl.ds(off, 16)]
            sv = src_v[pl.ds(off, 16)]
            plsc.addupdate_scatter(deg_l, [dv], ones)
            slv = plsc.load_gather(slot_v, [dv])
            m = slv >= 0
            packed = sv | lax.shift_left(slv, SLOT_SH)
            plsc.store_compressed(pk_v.at[pl.ds(cnt, 16)], packed, mask=m)
            npop = plsc.all_reduce_population_count(m)
            return cnt + npop[0]

        cnt = lax.fori_loop(0, EPW // 16, body, 0)
        padvec = jnp.full((16,), PADW, jnp.int32)
        for kk in range(CHUNK // 16):
            pk_v[pl.ds(cnt + kk * 16, 16)] = padvec
        cnt_v[...] = lax.broadcast(cnt, (16,))
        pltpu.sync_copy(cnt_v, cnt_hbm.at[wid])
        ngroups = lax.div(cnt + CHUNK - 1, CHUNK)

        def wgroup(g, carry):
            pltpu.sync_copy(pk_v.at[pl.ds(g * CHUNK, CHUNK)],
                            pk_hbm.at[wid].at[g])
            return carry

        lax.fori_loop(0, ngroups, wgroup, 0)

        pltpu.sync_copy(deg_l, stage_sh.at[sid])
        plsc.subcore_barrier()
        pltpu.sync_copy(stage_sh.at[0].at[pl.ds(sid * rpt, rpt)], acc_v)

        def red(p, carry):
            pltpu.sync_copy(stage_sh.at[p].at[pl.ds(sid * rpt, rpt)], tmp_v)

            def add16(j, c):
                off = pl.multiple_of(j * 16, 16)
                acc_v[pl.ds(off, 16)] = acc_v[pl.ds(off, 16)] + tmp_v[pl.ds(off, 16)]
                return c

            lax.fori_loop(0, rpt // 16, add16, 0)
            return carry

        lax.fori_loop(1, NS, red, 0)
        pltpu.sync_copy(acc_v, deg_hbm.at[cid].at[pl.ds(sid * rpt, rpt)])

    return k


def _prep_body(x_ref, w_ref, d0_ref, d1_ref, dinv_ref, hs_ref):
    h1 = jnp.dot(x_ref[...], w_ref[...], preferred_element_type=jnp.float32)
    deg = d0_ref[...] + d1_ref[...] + 1.0
    dinv = lax.rsqrt(deg)
    dinv_ref[...] = dinv
    hs_ref[pl.ds(0, NN), :] = h1 * dinv[:NN]
    hs_ref[pl.ds(NN, NP - NN), :] = jnp.zeros((NP - NN, H), jnp.float32)


def _layer2_body(a0_ref, a1_ref, hs_ref, dinv_ref, w2_ref, b1_ref, o_ref):
    dinv = dinv_ref[...]
    out1 = dinv * (a0_ref[...] + a1_ref[...] + hs_ref[...]) + b1_ref[...]
    r = jnp.maximum(out1, 0.0)
    h2 = jnp.dot(r, w2_ref[...], preferred_element_type=jnp.float32)
    o_ref[...] = h2 * dinv


def _final_body(a_ref, hsc_ref, dinvc_ref, b2_ref, o_ref):
    acc = jnp.sum(a_ref[...], axis=0)
    t = dinvc_ref[...] * (acc + hsc_ref[...]) + b2_ref[...]
    col = lax.broadcasted_iota(jnp.int32, t.shape, 1)
    logit = jnp.where(col < NCLS, t, -1e30)
    m = jnp.max(logit, axis=1, keepdims=True)
    e = jnp.exp(logit - m)
    s = jnp.sum(e, axis=1, keepdims=True)
    o_ref[...] = logit - m - jnp.log(s)


_edge_acc16 = _make_edge_acc(H)
_center_acc = _make_center_acc()
_deg_compact = _make_deg_compact()


def kernel(x, edge_index, batch, W1, b1, W2, b2):
    f32 = jnp.float32
    i32 = jnp.int32
    ei = edge_index.astype(i32)
    padi = jnp.full((EP - E,), PN, i32)
    srcp = jnp.concatenate([ei[0], padi])
    dstp = jnp.concatenate([ei[1], padi])
    src3 = srcp.reshape(NT, GPT, CHUNK)
    dst3 = dstp.reshape(NT, GPT, CHUNK)
    src2 = srcp.reshape(NT, GPT * CHUNK)
    dst2 = dstp.reshape(NT, GPT * CHUNK)
    zdeg = jnp.zeros((NP,), f32)
    z16 = jnp.zeros((NP // NS, H), f32)
    z48 = jnp.zeros((NP // NS, CPAD), f32)

    batch_i = batch.astype(i32)
    first = jnp.concatenate(
        [jnp.ones((1,), jnp.bool_), batch_i[1:] != batch_i[:-1]])
    slot_map = jnp.concatenate(
        [jnp.where(first, batch_i, -1), jnp.full((NP - NN,), -1, i32)])

    deg, pk, cnts = _deg_compact(src2, dst2, slot_map, zdeg)
    d0 = deg[0].reshape(NP, 1)
    d1 = deg[1].reshape(NP, 1)

    dinv, h1s = pl.pallas_call(
        _prep_body,
        out_shape=(jax.ShapeDtypeStruct((NP, 1), f32),
                   jax.ShapeDtypeStruct((NP, H), f32)),
    )(x, W1, d0, d1)

    acc1 = _edge_acc16(src3, dst3, h1s, z16)

    W2p = jnp.zeros((H, CPAD), f32).at[:, :NCLS].set(W2)
    h2s = pl.pallas_call(
        _layer2_body,
        out_shape=jax.ShapeDtypeStruct((NP, CPAD), f32),
    )(acc1[0], acc1[1], h1s, dinv, W2p, b1.reshape(1, H))

    acc2 = _center_acc(pk, cnts, h2s, z48)
    acc2r = acc2.reshape(NT, NSLOT, CPAD)

    hsc = h2s.reshape(NSLOT, NP // NSLOT, CPAD)[:, 0, :]
    dinvc = dinv.reshape(NSLOT, NP // NSLOT)[:, 0:1]
    b2p = jnp.zeros((1, CPAD), f32).at[0, :NCLS].set(b2)

    outp = pl.pallas_call(
        _final_body,
        out_shape=jax.ShapeDtypeStruct((NSLOT, CPAD), f32),
    )(acc2r, hsc, dinvc, b2p)
    return outp[:NGRAPH, :NCLS]


# speedup vs baseline: 1.0169x; 1.0169x over previous
import functools

import jax
import jax.numpy as jnp
from jax import lax
from jax.experimental import pallas as pl
from jax.experimental.pallas import tpu as pltpu
from jax.experimental.pallas import tpu_sc as plsc

NN = 10000
NP = 10240
PN = NN
E = 320000
NCLS = 40
H = 16
CPAD = 48
NC = 2
NS = 16
NT = NC * NS
CHUNK = 128
GPT = 79
EP = NT * GPT * CHUNK
DEGR = 80
NGRAPH = 250
NSLOT = 256
SLOT_SH = 18


def _mesh():
    return plsc.VectorSubcoreMesh(core_axis_name="c", subcore_axis_name="s")


def _make_edge_acc(width):
    rpt = NP // NS

    @functools.partial(
        pl.kernel,
        out_type=jax.ShapeDtypeStruct((NC, NP, width), jnp.float32),
        mesh=_mesh(),
        scratch_types=[
            pltpu.VMEM((GPT, CHUNK), jnp.int32),
            pltpu.VMEM((GPT, CHUNK), jnp.int32),
            pltpu.VMEM((CHUNK, width), jnp.float32),
            pltpu.VMEM((CHUNK, width), jnp.float32),
            pltpu.VMEM_SHARED((NP, width), jnp.float32),
            pltpu.SemaphoreType.DMA,
            pltpu.SemaphoreType.DMA,
        ],
        compiler_params=pltpu.CompilerParams(use_tc_tiling_on_sc=False),
    )
    def k(src_hbm, dst_hbm, feat_hbm, zrows_hbm, out_hbm,
          src_v, dst_v, rows_a, rows_b, acc_sh, sem_a, sem_b):
        cid = lax.axis_index("c")
        sid = lax.axis_index("s")
        wid = sid * NC + cid
        pltpu.sync_copy(zrows_hbm, acc_sh.at[pl.ds(sid * rpt, rpt)])
        pltpu.sync_copy(src_hbm.at[wid], src_v)
        pltpu.sync_copy(dst_hbm.at[wid], dst_v)
        plsc.subcore_barrier()
        pltpu.async_copy(feat_hbm.at[src_v.at[0]], rows_a, sem_a)

        def body(i, carry):
            ga = 2 * i
            pltpu.async_copy(feat_hbm.at[src_v.at[ga + 1]], rows_b, sem_b)
            pltpu.make_async_copy(feat_hbm.at[src_v.at[ga]], rows_a, sem_a).wait()
            pltpu.sync_copy(rows_a, acc_sh.at[dst_v.at[ga]], add=True)
            pltpu.async_copy(feat_hbm.at[src_v.at[ga + 2]], rows_a, sem_a)
            pltpu.make_async_copy(feat_hbm.at[src_v.at[ga + 1]], rows_b, sem_b).wait()
            pltpu.sync_copy(rows_b, acc_sh.at[dst_v.at[ga + 1]], add=True)
            return carry

        lax.fori_loop(0, (GPT - 1) // 2, body, 0)
        pltpu.make_async_copy(feat_hbm.at[src_v.at[GPT - 1]], rows_a, sem_a).wait()
        pltpu.sync_copy(rows_a, acc_sh.at[dst_v.at[GPT - 1]], add=True)
        plsc.subcore_barrier()
        pltpu.sync_copy(acc_sh.at[pl.ds(sid * rpt, rpt)],
                        out_hbm.at[cid].at[pl.ds(sid * rpt, rpt)])

    return k


def _make_center_acc():
    @functools.partial(
        pl.kernel,
        out_type=jax.ShapeDtypeStruct((NC, NSLOT, CPAD), jnp.float32),
        mesh=_mesh(),
        scratch_types=[
            pltpu.VMEM((16,), jnp.int32),
            pltpu.VMEM((CHUNK,), jnp.int32),
            pltpu.VMEM((CHUNK,), jnp.int32),
            pltpu.VMEM((CHUNK,), jnp.int32),
            pltpu.VMEM((CHUNK, CPAD), jnp.float32),
            pltpu.VMEM_SHARED((NSLOT, CPAD), jnp.float32),
            pltpu.SemaphoreType.DMA,
        ],
        compiler_params=pltpu.CompilerParams(
            use_tc_tiling_on_sc=False, needs_layout_passes=False),
    )
    def k(pk_hbm, cnt_hbm, feat_hbm, zrows_hbm, out_hbm,
          cnt_v, pkg_v, gidx_v, sidx_v, rows_v, acc_sh, sem):
        cid = lax.axis_index("c")
        sid = lax.axis_index("s")
        wid = sid * NC + cid
        pltpu.sync_copy(zrows_hbm.at[pl.ds(0, NSLOT // NS)],
                        acc_sh.at[pl.ds(sid * (NSLOT // NS), NSLOT // NS)])
        pltpu.sync_copy(cnt_hbm.at[wid], cnt_v)
        plsc.subcore_barrier()
        cnt = cnt_v[pl.ds(0, 16)][0]
        ngroups = lax.div(cnt + CHUNK - 1, CHUNK)

        def group(g, carry):
            pltpu.sync_copy(pk_hbm.at[wid].at[g], pkg_v)
            for j in range(CHUNK // 16):
                p = pkg_v[pl.ds(j * 16, 16)]
                gidx_v[pl.ds(j * 16, 16)] = jnp.bitwise_and(p, (1 << SLOT_SH) - 1)
                sidx_v[pl.ds(j * 16, 16)] = lax.shift_right_logical(p, SLOT_SH)
            pltpu.async_copy(feat_hbm.at[gidx_v], rows_v, sem).wait()
            pltpu.sync_copy(rows_v, acc_sh.at[sidx_v], add=True)
            return carry

        lax.fori_loop(0, ngroups, group, 0)
        plsc.subcore_barrier()
        pltpu.sync_copy(acc_sh.at[pl.ds(sid * (NSLOT // NS), NSLOT // NS)],
                        out_hbm.at[cid].at[pl.ds(sid * (NSLOT // NS), NSLOT // NS)])

    return k


def _make_deg_compact():
    rpt = NP // NS
    EPW = GPT * CHUNK
    PADW = PN | (NGRAPH << SLOT_SH)

    @functools.partial(
        pl.kernel,
        out_type=(jax.ShapeDtypeStruct((NC, NP), jnp.float32),
                  jax.ShapeDtypeStruct((NT, GPT + 1, CHUNK), jnp.int32),
                  jax.ShapeDtypeStruct((NT, 16), jnp.int32)),
        mesh=_mesh(),
        scratch_types=[
            pltpu.VMEM((EPW,), jnp.int32),
            pltpu.VMEM((EPW,), jnp.int32),
            pltpu.VMEM((NP,), jnp.int32),
            pltpu.VMEM((EPW + CHUNK,), jnp.int32),
            pltpu.VMEM((16,), jnp.int32),
            pltpu.VMEM((NP,), jnp.float32),
            pltpu.VMEM((rpt,), jnp.float32),
            pltpu.VMEM((rpt,), jnp.float32),
            pltpu.VMEM_SHARED((NS, NP), jnp.float32),
        ],
        compiler_params=pltpu.CompilerParams(
            use_tc_tiling_on_sc=False, needs_layout_passes=False),
    )
    def k(src_hbm, dst_hbm, slotmap_hbm, zdeg_hbm,
          deg_hbm, pk_hbm, cnt_hbm,
          src_v, dst_v, slot_v, pk_v, cnt_v, deg_l, acc_v, tmp_v, stage_sh):
        cid = lax.axis_index("c")
        sid = lax.axis_index("s")
        wid = sid * NC + cid
        pltpu.sync_copy(zdeg_hbm, deg_l)
        pltpu.sync_copy(src_hbm.at[wid], src_v)
        pltpu.sync_copy(dst_hbm.at[wid], dst_v)
        pltpu.sync_copy(slotmap_hbm, slot_v)
        ones = jnp.full((16,), 1.0, jnp.float32)

        def body(i, cnt):
            off = pl.multiple_of(i * 16, 16)
            dv = dst_v[pl.ds(off, 16)]
            sv = src_v[pl.ds(off, 16)]
            plsc.addupdate_scatter(deg_l, [dv], ones)
            slv = plsc.load_gather(slot_v, [dv])
            m = slv >= 0
            packed = sv | lax.shift_left(slv, SLOT_SH)
            plsc.store_compressed(pk_v.at[pl.ds(cnt, 16)], packed, mask=m)
            npop = plsc.all_reduce_population_count(m)
            return cnt + npop[0]

        cnt = lax.fori_loop(0, EPW // 16, body, 0)
        padvec = jnp.full((16,), PADW, jnp.int32)
        for kk in range(CHUNK // 16):
            pk_v[pl.ds(cnt + kk * 16, 16)] = padvec
        cnt_v[...] = lax.broadcast(cnt, (16,))
        pltpu.sync_copy(cnt_v, cnt_hbm.at[wid])
        ngroups = lax.div(cnt + CHUNK - 1, CHUNK)

        def wgroup(g, carry):
            pltpu.sync_copy(pk_v.at[pl.ds(g * CHUNK, CHUNK)],
                            pk_hbm.at[wid].at[g])
            return carry

        lax.fori_loop(0, ngroups, wgroup, 0)

        pltpu.sync_copy(deg_l, stage_sh.at[sid])
        plsc.subcore_barrier()
        pltpu.sync_copy(stage_sh.at[0].at[pl.ds(sid * rpt, rpt)], acc_v)

        def red(p, carry):
            pltpu.sync_copy(stage_sh.at[p].at[pl.ds(sid * rpt, rpt)], tmp_v)

            def add16(j, c):
                off = pl.multiple_of(j * 16, 16)
                acc_v[pl.ds(off, 16)] = acc_v[pl.ds(off, 16)] + tmp_v[pl.ds(off, 16)]
                return c

            lax.fori_loop(0, rpt // 16, add16, 0)
            return carry

        lax.fori_loop(1, NS, red, 0)
        pltpu.sync_copy(acc_v, deg_hbm.at[cid].at[pl.ds(sid * rpt, rpt)])

    return k


def _prep_body(x_ref, w_ref, d0_ref, d1_ref, dinv_ref, hs_ref):
    h1 = jnp.dot(x_ref[...], w_ref[...], preferred_element_type=jnp.float32)
    deg = d0_ref[...] + d1_ref[...] + 1.0
    dinv = lax.rsqrt(deg)
    dinv_ref[...] = dinv
    hs_ref[pl.ds(0, NN), :] = h1 * dinv[:NN]
    hs_ref[pl.ds(NN, NP - NN), :] = jnp.zeros((NP - NN, H), jnp.float32)


def _layer2_body(a0_ref, a1_ref, hs_ref, dinv_ref, w2_ref, b1_ref, o_ref):
    dinv = dinv_ref[...]
    out1 = dinv * (a0_ref[...] + a1_ref[...] + hs_ref[...]) + b1_ref[...]
    r = jnp.maximum(out1, 0.0)
    h2 = jnp.dot(r, w2_ref[...], preferred_element_type=jnp.float32)
    o_ref[...] = h2 * dinv


def _final_body(a0_ref, a1_ref, hsc_ref, dinvc_ref, b2_ref, o_ref):
    t = dinvc_ref[...] * (a0_ref[...] + a1_ref[...] + hsc_ref[...]) + b2_ref[...]
    col = lax.broadcasted_iota(jnp.int32, t.shape, 1)
    logit = jnp.where(col < NCLS, t, -1e30)
    m = jnp.max(logit, axis=1, keepdims=True)
    e = jnp.exp(logit - m)
    s = jnp.sum(e, axis=1, keepdims=True)
    o_ref[...] = logit - m - jnp.log(s)


_edge_acc16 = _make_edge_acc(H)
_center_acc = _make_center_acc()
_deg_compact = _make_deg_compact()


def kernel(x, edge_index, batch, W1, b1, W2, b2):
    f32 = jnp.float32
    i32 = jnp.int32
    ei = edge_index.astype(i32)
    padi = jnp.full((EP - E,), PN, i32)
    srcp = jnp.concatenate([ei[0], padi])
    dstp = jnp.concatenate([ei[1], padi])
    src3 = srcp.reshape(NT, GPT, CHUNK)
    dst3 = dstp.reshape(NT, GPT, CHUNK)
    src2 = srcp.reshape(NT, GPT * CHUNK)
    dst2 = dstp.reshape(NT, GPT * CHUNK)
    zdeg = jnp.zeros((NP,), f32)
    z16 = jnp.zeros((NP // NS, H), f32)
    z48 = jnp.zeros((NP // NS, CPAD), f32)

    batch_i = batch.astype(i32)
    first = jnp.concatenate(
        [jnp.ones((1,), jnp.bool_), batch_i[1:] != batch_i[:-1]])
    slot_map = jnp.concatenate(
        [jnp.where(first, batch_i, -1), jnp.full((NP - NN,), -1, i32)])

    deg, pk, cnts = _deg_compact(src2, dst2, slot_map, zdeg)
    d0 = deg[0].reshape(NP, 1)
    d1 = deg[1].reshape(NP, 1)

    dinv, h1s = pl.pallas_call(
        _prep_body,
        out_shape=(jax.ShapeDtypeStruct((NP, 1), f32),
                   jax.ShapeDtypeStruct((NP, H), f32)),
    )(x, W1, d0, d1)

    acc1 = _edge_acc16(src3, dst3, h1s, z16)

    W2p = jnp.zeros((H, CPAD), f32).at[:, :NCLS].set(W2)
    h2s = pl.pallas_call(
        _layer2_body,
        out_shape=jax.ShapeDtypeStruct((NP, CPAD), f32),
    )(acc1[0], acc1[1], h1s, dinv, W2p, b1.reshape(1, H))

    acc2 = _center_acc(pk, cnts, h2s, z48)

    hsc = h2s.reshape(NSLOT, NP // NSLOT, CPAD)[:, 0, :]
    dinvc = dinv.reshape(NSLOT, NP // NSLOT)[:, 0:1]
    b2p = jnp.zeros((1, CPAD), f32).at[0, :NCLS].set(b2)

    outp = pl.pallas_call(
        _final_body,
        out_shape=jax.ShapeDtypeStruct((NSLOT, CPAD), f32),
    )(acc2[0], acc2[1], hsc, dinvc, b2p)
    return outp[:NGRAPH, :NCLS]
